# baseline (speedup 1.0000x reference)
import functools

import jax
import jax.numpy as jnp
from jax import lax
from jax.experimental import pallas as pl
from jax.experimental.pallas import tpu as pltpu
from jax.experimental.pallas import tpu_sc as plsc

BATCH = 16384
HIST = 50
EMBED_DIM = 32
TOTAL = BATCH * HIST
VOCAB = 1000000

NUM_CORES = 2
NUM_WORKERS = 32

CHUNK = 256
CHUNKS_PER_H = BATCH // CHUNK
NUM_UNITS = TOTAL // CHUNK
UNITS_PER_WORKER = NUM_UNITS // NUM_WORKERS

_mesh = plsc.VectorSubcoreMesh(core_axis_name="c", subcore_axis_name="s")


@functools.partial(
    pl.kernel,
    mesh=_mesh,
    out_type=jax.ShapeDtypeStruct((HIST, EMBED_DIM, BATCH), jnp.float32),
    scratch_types=[
        pltpu.VMEM((CHUNK,), jnp.int32),
        pltpu.VMEM((CHUNK, EMBED_DIM), jnp.float32),
        pltpu.VMEM((EMBED_DIM * CHUNK,), jnp.float32),
        pltpu.SemaphoreType.DMA,
        pltpu.SemaphoreType.DMA,
    ],
    compiler_params=pltpu.CompilerParams(
        use_tc_tiling_on_sc=False, needs_layout_passes=False),
)
def _gather_kernel(idx_hbm, table_hbm, out_hbm, idx_v, rows_v, tbuf,
                   gsem, osem):
    wid = lax.axis_index("s") * NUM_CORES + lax.axis_index("c")
    iota16 = jax.lax.iota(jnp.int32, 16)
    iota_sc = iota16 * CHUNK
    table32 = table_hbm

    def body(k, carry):
        u = k * NUM_WORKERS + wid
        off = u * CHUNK
        h = u // CHUNKS_PER_H
        b0 = (u % CHUNKS_PER_H) * CHUNK

        pltpu.sync_copy(idx_hbm.at[pl.ds(off, CHUNK)], idx_v)
        pltpu.async_copy(table32.at[idx_v], rows_v, gsem).wait()

        for r in range(CHUNK):
            for q in range(EMBED_DIM // 16):
                vals = rows_v[r, pl.ds(q * 16, 16)]
                plsc.store_scatter(
                    tbuf, [iota_sc + (q * 16 * CHUNK + r)], vals)

        copies = [
            pltpu.async_copy(
                tbuf.at[pl.ds(c * CHUNK, CHUNK)],
                out_hbm.at[h, c, pl.ds(b0, CHUNK)],
                osem,
            )
            for c in range(EMBED_DIM)
        ]
        for cp in copies:
            cp.wait()
        return carry

    lax.fori_loop(0, UNITS_PER_WORKER, body, 0)


def kernel(inputs, table):
    idx_h_major = inputs.T.reshape(TOTAL)
    out = _gather_kernel(idx_h_major, table)
    return out.transpose(2, 0, 1)


# speedup vs baseline: 1.0992x; 1.0992x over previous
import functools

import jax
import jax.numpy as jnp
from jax import lax
from jax.experimental import pallas as pl
from jax.experimental.pallas import tpu as pltpu
from jax.experimental.pallas import tpu_sc as plsc

BATCH = 16384
HIST = 50
EMBED_DIM = 32
TOTAL = BATCH * HIST
VOCAB = 1000000

NUM_CORES = 2
NUM_WORKERS = 32

CHUNK = 512
CHUNKS_PER_H = BATCH // CHUNK
NUM_UNITS = TOTAL // CHUNK
UNITS_PER_WORKER = NUM_UNITS // NUM_WORKERS

_mesh = plsc.VectorSubcoreMesh(core_axis_name="c", subcore_axis_name="s")


@functools.partial(
    pl.kernel,
    mesh=_mesh,
    out_type=jax.ShapeDtypeStruct((HIST, EMBED_DIM, BATCH), jnp.float32),
    scratch_types=[
        pltpu.VMEM((CHUNK,), jnp.int32),
        pltpu.VMEM((CHUNK,), jnp.int32),
        pltpu.VMEM((CHUNK, EMBED_DIM), jnp.float32),
        pltpu.VMEM((CHUNK, EMBED_DIM), jnp.float32),
        pltpu.VMEM((EMBED_DIM * CHUNK,), jnp.float32),
        pltpu.VMEM((EMBED_DIM * CHUNK,), jnp.float32),
        pltpu.SemaphoreType.DMA,
        pltpu.SemaphoreType.DMA,
        pltpu.SemaphoreType.DMA,
    ],
    compiler_params=pltpu.CompilerParams(
        use_tc_tiling_on_sc=False, needs_layout_passes=False),
)
def _gather_kernel(idx_hbm, table_hbm, out_hbm, idxa, idxb, rowsa, rowsb,
                   ta, tb, gsema, gsemb, osem):
    wid = lax.axis_index("s") * NUM_CORES + lax.axis_index("c")
    iota_sc = jax.lax.iota(jnp.int32, 16) * CHUNK

    def fetch(k, idx_v, rows_v, gsem):
        off = (k * NUM_WORKERS + wid) * CHUNK
        pltpu.sync_copy(idx_hbm.at[pl.ds(off, CHUNK)], idx_v)
        return pltpu.async_copy(table_hbm.at[idx_v], rows_v, gsem)

    def process(k, rows_v, tbuf, gsem):
        pltpu.make_async_copy(table_hbm.at[pl.ds(0, CHUNK)], rows_v,
                              gsem).wait()
        for r in range(CHUNK):
            for q in range(EMBED_DIM // 16):
                vals = rows_v[r, pl.ds(q * 16, 16)]
                plsc.store_scatter(
                    tbuf, [iota_sc + (q * 16 * CHUNK + r)], vals)
        u = k * NUM_WORKERS + wid
        h = u // CHUNKS_PER_H
        b0 = (u % CHUNKS_PER_H) * CHUNK
        copies = [
            pltpu.async_copy(
                tbuf.at[pl.ds(c * CHUNK, CHUNK)],
                out_hbm.at[h, c, pl.ds(b0, CHUNK)],
                osem,
            )
            for c in range(EMBED_DIM)
        ]
        for cp in copies:
            cp.wait()

    fetch(0, idxa, rowsa, gsema)

    def body(k2, carry):
        ka = 2 * k2
        fetch(ka + 1, idxb, rowsb, gsemb)
        process(ka, rowsa, ta, gsema)

        @pl.when(k2 + 1 < UNITS_PER_WORKER // 2)
        def _():
            fetch(ka + 2, idxa, rowsa, gsema)

        process(ka + 1, rowsb, tb, gsemb)
        return carry

    lax.fori_loop(0, UNITS_PER_WORKER // 2, body, 0)


def kernel(inputs, table):
    idx_h_major = inputs.T.reshape(TOTAL)
    out = _gather_kernel(idx_h_major, table)
    return out.transpose(2, 0, 1)
